# baseline (speedup 1.0000x reference)
import functools

import jax
import jax.numpy as jnp
import numpy as np
from jax import lax
from jax.experimental import pallas as pl
from jax.experimental.pallas import tpu as pltpu
from jax.experimental.pallas import tpu_sc as plsc

_INT_MIN = np.int32(-(2**31))


def _scores_body(x_ref, w1_ref, b1_ref, w2t_ref, b2_ref, s_ref):
    h = jnp.dot(x_ref[...], w1_ref[...], preferred_element_type=jnp.float32)
    h = h + b1_ref[...]
    h = 0.5 * h * (1.0 + lax.erf(h * np.float32(0.7071067811865476)))
    s = jnp.sum(h * w2t_ref[...], axis=1) + b2_ref[0, 0]
    s_ref[...] = s


def _select_body(keep_n, s_ref, a_ref, t_ref):
    n = s_ref.shape[1]

    @pl.when(pl.program_id(0) == 0)
    def _():
        r = lax.broadcasted_iota(jnp.int32, (n, n), 0)
        c = lax.broadcasted_iota(jnp.int32, (n, n), 1)
        t_ref[...] = (r <= c).astype(jnp.float32)

    s = s_ref[...]
    bits = lax.bitcast_convert_type(s, jnp.int32)
    key = jnp.where(bits >= 0, bits, bits ^ np.int32(0x7FFFFFFF))
    bb = s.shape[0]
    cu = jnp.zeros((bb, 1), jnp.int32)
    for bit in range(31, -1, -1):
        bitval = _INT_MIN if bit == 31 else np.int32(1 << bit)
        cand = cu | bitval
        cand_s = cand ^ _INT_MIN
        cnt = jnp.sum((key >= cand_s).astype(jnp.int32), axis=1, keepdims=True)
        cu = jnp.where(cnt >= keep_n, cand, cu)
    t_s = cu ^ _INT_MIN

    gt = key > t_s
    eq = key == t_s
    c_gt = jnp.sum(gt.astype(jnp.int32), axis=1, keepdims=True)
    need = (keep_n - c_gt).astype(jnp.float32)
    eqf = eq.astype(jnp.float32)
    eqcum = jnp.dot(eqf, t_ref[...], preferred_element_type=jnp.float32)
    tie = eq & ((eqcum - eqf) < need)
    keepm = gt | tie
    keepf = keepm.astype(jnp.float32)
    dcum = jnp.dot(keepf, t_ref[...], preferred_element_type=jnp.float32)
    dest = dcum.astype(jnp.int32) - 1
    a_ref[...] = jnp.where(keepm, dest, np.int32(-1))


def _sc_gather_body(n, d, keep, x_hbm, a_hbm, out_hbm, spm, dest_v, idx_v, rows_v, sem):
    cid = lax.axis_index("c")
    sid = lax.axis_index("s")
    iota = lax.iota(jnp.int32, 16)
    ngrp = 16

    def grp_body(g, carry):
        @pl.when(sid < 8)
        def _():
            b = cid * 128 + g * 8 + sid
            pltpu.sync_copy(x_hbm.at[pl.ds(b * n, n)], spm.at[sid])

        plsc.subcore_barrier()
        r = sid // 2
        h = sid % 2
        b = cid * 128 + g * 8 + r
        pltpu.sync_copy(a_hbm.at[b], dest_v)

        def zbody(c, carry2):
            idx_v[pl.ds(c * 16, 16)] = jnp.zeros((16,), jnp.int32)
            return carry2

        lax.fori_loop(0, 24, zbody, 0)

        def cbody(c, carry2):
            dchunk = dest_v[pl.ds(c * 16, 16)]
            dl = dchunk - h * 384
            msk = (dl >= 0) & (dl < 384)
            dsafe = jnp.where(msk, dl, 0)
            src = iota + c * 16
            plsc.store_scatter(idx_v, [dsafe], src, mask=msk)
            return carry2

        lax.fori_loop(0, n // 16, cbody, 0)

        def gbody(c, carry2):
            pltpu.async_copy(
                spm.at[r].at[idx_v.at[pl.ds(c * 128, 128)]], rows_v, sem
            ).wait()
            pltpu.sync_copy(rows_v, out_hbm.at[b, h * 3 + c])
            return carry2

        lax.fori_loop(0, 3, gbody, 0)
        plsc.subcore_barrier()
        return carry

    lax.fori_loop(0, ngrp, grp_body, 0)


def kernel(x, W1, b1, W2, b2):
    B, N, D = x.shape
    H = W1.shape[1]
    KEEP = (N * 7) // 10
    CH = 120
    NCH = -(-KEEP // CH)
    assert NCH * CH >= KEEP and N % 16 == 0

    x2d = x.reshape(B * N, D)
    R1 = 8192
    scores = pl.pallas_call(
        _scores_body,
        grid=(B * N // R1,),
        in_specs=[
            pl.BlockSpec((R1, D), lambda i: (i, 0)),
            pl.BlockSpec((D, H), lambda i: (0, 0)),
            pl.BlockSpec((1, H), lambda i: (0, 0)),
            pl.BlockSpec((1, H), lambda i: (0, 0)),
            pl.BlockSpec((1, 1), lambda i: (0, 0)),
        ],
        out_specs=pl.BlockSpec((R1,), lambda i: (i,)),
        out_shape=jax.ShapeDtypeStruct((B * N,), jnp.float32),
    )(x2d, W1, b1.reshape(1, H), W2.reshape(1, H), b2.reshape(1, 1))

    BB = 32
    A = pl.pallas_call(
        functools.partial(_select_body, KEEP),
        grid=(B // BB,),
        in_specs=[pl.BlockSpec((BB, N), lambda i: (i, 0))],
        out_specs=pl.BlockSpec((BB, N), lambda i: (i, 0)),
        out_shape=jax.ShapeDtypeStruct((B, N), jnp.int32),
        scratch_shapes=[pltpu.VMEM((N, N), jnp.float32)],
    )(scores.reshape(B, N))

    info = plsc.get_sparse_core_info()
    assert info.num_cores == 2 and info.num_subcores == 16

    sc_fn = functools.partial(_sc_gather_body, N, D, KEEP)
    out4 = pl.kernel(
        sc_fn,
        mesh=plsc.VectorSubcoreMesh(core_axis_name="c", subcore_axis_name="s"),
        compiler_params=pltpu.CompilerParams(needs_layout_passes=False),
        out_type=jax.ShapeDtypeStruct((B, 6, 128, D), jnp.float32),
        scratch_types=[
            pltpu.VMEM_SHARED((8, N, D), jnp.float32),
            pltpu.VMEM((N,), jnp.int32),
            pltpu.VMEM((3 * 128,), jnp.int32),
            pltpu.VMEM((128, D), jnp.float32),
            pltpu.SemaphoreType.DMA,
        ],
    )(x2d, A)

    return out4.reshape(B, 6 * 128, D)[:, :KEEP, :]
